# baseline (speedup 1.0000x reference)
import functools

import jax
import jax.numpy as jnp
from jax.experimental import pallas as pl
from jax.experimental.pallas import tpu as pltpu

_VMEM_LIMIT = 100 * 1024 * 1024


def _mlp_kernel(x_ref, w0_ref, b0_ref, w1_ref, b1_ref, w2_ref, b2_ref, o_ref):
    t = x_ref[...].astype(jnp.bfloat16)
    t = jnp.dot(t, w0_ref[...], preferred_element_type=jnp.float32) + b0_ref[...]
    t = jnp.maximum(t, 0.0).astype(jnp.bfloat16)
    t = jnp.dot(t, w1_ref[...], preferred_element_type=jnp.float32) + b1_ref[...]
    t = jnp.maximum(t, 0.0).astype(jnp.bfloat16)
    t = jnp.dot(t, w2_ref[...], preferred_element_type=jnp.float32) + b2_ref[...]
    o_ref[...] = jnp.maximum(t, 0.0)


def _mlp(x, w0, b0, w1, b1, w2, b2, *, tm):
    n, fin = x.shape
    f0, f1, f2 = w0.shape[1], w1.shape[1], w2.shape[1]
    tm = min(tm, n)
    grid = (n // tm,)
    return pl.pallas_call(
        _mlp_kernel,
        out_shape=jax.ShapeDtypeStruct((n, f2), jnp.float32),
        grid=grid,
        in_specs=[
            pl.BlockSpec((tm, fin), lambda i: (i, 0)),
            pl.BlockSpec((fin, f0), lambda i: (0, 0)),
            pl.BlockSpec((1, f0), lambda i: (0, 0)),
            pl.BlockSpec((f0, f1), lambda i: (0, 0)),
            pl.BlockSpec((1, f1), lambda i: (0, 0)),
            pl.BlockSpec((f1, f2), lambda i: (0, 0)),
            pl.BlockSpec((1, f2), lambda i: (0, 0)),
        ],
        out_specs=pl.BlockSpec((tm, f2), lambda i: (i, 0)),
        compiler_params=pltpu.CompilerParams(
            dimension_semantics=("parallel",),
            vmem_limit_bytes=_VMEM_LIMIT,
        ),
    )(x, w0, b0, w1, b1, w2, b2)


def _prop_kernel(c_ref, h_ref, dinv_full_ref, dinv_blk_ref, x0_ref, o_ref,
                 *, alpha, tm, nsub):
    g = h_ref[...] * dinv_full_ref[...]
    acc = jnp.dot(c_ref[0:tm, :], g[0:128, :],
                  preferred_element_type=jnp.float32)
    for k in range(1, nsub):
        acc += jnp.dot(c_ref[k * tm:(k + 1) * tm, :],
                       g[k * 128:(k + 1) * 128, :],
                       preferred_element_type=jnp.float32)
    i = pl.program_id(0)
    g_blk = h_ref[pl.ds(i * tm, tm), :] * dinv_blk_ref[...]
    o_ref[...] = ((1.0 - alpha) * dinv_blk_ref[...] * (acc + g_blk)
                  + alpha * x0_ref[...])


def _prop_step(counts, h, dinv, x0, *, alpha, tm):
    n, f = x0.shape
    tm = min(tm, n)
    nsub = n // 128
    grid = (n // tm,)
    return pl.pallas_call(
        functools.partial(_prop_kernel, alpha=alpha, tm=tm, nsub=nsub),
        out_shape=jax.ShapeDtypeStruct((n, f), jnp.float32),
        grid=grid,
        in_specs=[
            pl.BlockSpec((tm * nsub, 128), lambda i: (i, 0)),
            pl.BlockSpec((n, f), lambda i: (0, 0)),
            pl.BlockSpec((n, 1), lambda i: (0, 0)),
            pl.BlockSpec((tm, 1), lambda i: (i, 0)),
            pl.BlockSpec((tm, f), lambda i: (i, 0)),
        ],
        out_specs=pl.BlockSpec((tm, f), lambda i: (i, 0)),
        compiler_params=pltpu.CompilerParams(
            dimension_semantics=("parallel",),
            vmem_limit_bytes=_VMEM_LIMIT,
        ),
    )(counts, h, dinv, dinv, x0)


def kernel(x, edge_index, w0, w1, w2, b0, b1, b2):
    n = x.shape[0]
    alpha = 0.1
    k_steps = 2

    x0 = _mlp(
        x,
        w0.astype(jnp.bfloat16), b0,
        w1.astype(jnp.bfloat16), b1,
        w2.astype(jnp.bfloat16), b2,
        tm=1024,
    )

    src = edge_index[0]
    dst = edge_index[1]
    tm = min(512, n)
    nsub = n // 128
    row = (dst // tm) * (tm * nsub) + (src // 128) * tm + (dst % tm)
    pos = row * 128 + (src % 128)
    ones = jnp.ones((dst.shape[0],), jnp.float32)
    flat = jnp.zeros((n * n,), jnp.float32).at[pos].add(ones)
    counts_slabs = flat.reshape(n * nsub, 128)
    deg = (counts_slabs.reshape(n // tm, nsub, tm, 128)
           .sum(axis=(1, 3)).reshape(n)) + 1.0
    dinv = jax.lax.rsqrt(deg)[:, None]

    h = x0
    for _ in range(k_steps):
        h = _prop_step(counts_slabs, h, dinv, x0, alpha=alpha, tm=tm)
    return h
